# baseline (speedup 1.0000x reference)
import functools

import jax
import jax.numpy as jnp
from jax import lax
from jax.experimental import pallas as pl
from jax.experimental.pallas import tpu as pltpu
from jax.experimental.pallas import tpu_sc as plsc

NUM_CORES = 2
NUM_SUBCORES = 16
NUM_WORKERS = NUM_CORES * NUM_SUBCORES

BATCH = 16384
HIST_LEN = 50
DIM = 64
VOCAB = 1000000
NPAIR = VOCAB // 2
NBLK = VOCAB // 128
TAIL_V = VOCAB - NBLK * 128
TAIL_W = NBLK % NUM_WORKERS
N_GROUPS1 = 123

BCOLS = BATCH // 128
BCOLS_PER_W = BCOLS // NUM_WORKERS
N_BLOCKS2 = HIST_LEN * BCOLS_PER_W
N_GROUPS2 = N_BLOCKS2 // 2

_MESH = plsc.VectorSubcoreMesh(
    core_axis_name="c",
    subcore_axis_name="s",
    num_cores=NUM_CORES,
    num_subcores=NUM_SUBCORES,
)

_PARAMS = pltpu.CompilerParams(use_tc_tiling_on_sc=True,
                               needs_layout_passes=False)

def _lane():
    return lax.iota(jnp.int32, 16)


def _transpose_pack(in_view, out_view, width):
    def body(p, _):
        col_a = jnp.broadcast_to(2 * p, (16,))
        col_b = col_a + 1
        for c16 in range(8):
            row_idx = _lane() + (16 * c16) % 64
            col_idx = col_a if c16 < 4 else col_b
            v = plsc.load_gather(in_view, [row_idx, col_idx])
            out_view[p, pl.ds(16 * c16, 16)] = v
        return 0

    lax.fori_loop(0, width // 2, body, 0)


@functools.partial(
    pl.kernel,
    out_type=jax.ShapeDtypeStruct((NPAIR, 128), jnp.float32),
    mesh=_MESH,
    scratch_types=(
        [pltpu.VMEM((DIM, 128), jnp.float32) for _ in range(2)]
        + [pltpu.VMEM((64, 128), jnp.float32) for _ in range(2)]
        + [pltpu.SemaphoreType.DMA for _ in range(4)]
    ),
    compiler_params=_PARAMS,
)
def _pack_kernel(tab_t, tail_t, out_hbm, in0, in1, ot0, ot1, sg0, sg1,
                 sw0, sw1):
    wid = lax.axis_index("s") * NUM_CORES + lax.axis_index("c")
    ins = [in0, in1]
    outs = [ot0, ot1]
    sem_g = [sg0, sg1]
    sem_w = [sw0, sw1]
    nk = (NBLK - 1 - wid) // NUM_WORKERS + 1

    def in_desc(k, b):
        blk = wid + k * NUM_WORKERS
        return pltpu.make_async_copy(
            tab_t.at[:, pl.ds(pl.multiple_of(blk * 128, 128), 128)],
            ins[b], sem_g[b])

    def out_desc(k, b):
        blk = wid + k * NUM_WORKERS
        return pltpu.make_async_copy(
            outs[b], out_hbm.at[pl.ds(pl.multiple_of(blk * 64, 64), 64)],
            sem_w[b])

    in_desc(0, 0).start()

    def group(g, _):
        for b in range(2):
            k = 2 * g + b

            @pl.when(k < nk)
            def _():
                @pl.when(k + 1 < nk)
                def _():
                    in_desc(k + 1, 1 - b).start()

                in_desc(k, b).wait()

                @pl.when(k >= 2)
                def _():
                    out_desc(k - 2, b).wait()

                _transpose_pack(ins[b], outs[b], 128)
                out_desc(k, b).start()
        return 0

    lax.fori_loop(0, N_GROUPS1, group, 0)

    for b in range(2):
        j = nk - 1 - lax.rem(nk - 1 + b, 2)
        out_desc(j, b).wait()

    @pl.when(wid == TAIL_W)
    def _():
        pltpu.sync_copy(tail_t, ins[0])
        _transpose_pack(ins[0], outs[0], TAIL_V)
        pltpu.sync_copy(outs[0].at[pl.ds(0, TAIL_V // 2)],
                        out_hbm.at[pl.ds(NBLK * 64, TAIL_V // 2)])


@functools.partial(
    pl.kernel,
    out_type=jax.ShapeDtypeStruct((HIST_LEN, DIM, BATCH), jnp.float32),
    mesh=_MESH,
    scratch_types=(
        [pltpu.VMEM((HIST_LEN, 512), jnp.int32)]
        + [pltpu.VMEM((128,), jnp.int32) for _ in range(2)]
        + [pltpu.VMEM((128, 128), jnp.float32) for _ in range(2)]
        + [pltpu.VMEM((1, DIM, 128), jnp.float32) for _ in range(2)]
        + [pltpu.SemaphoreType.DMA for _ in range(4)]
    ),
    compiler_params=_PARAMS,
)
def _gather_kernel(packed, x_t, out_hbm, idx_slab, pi0, pi1, rw0, rw1,
                   ot0, ot1, sg0, sg1, sw0, sw1):
    wid = lax.axis_index("s") * NUM_CORES + lax.axis_index("c")
    pidx = [pi0, pi1]
    rows = [rw0, rw1]
    outs = [ot0, ot1]
    sem_g = [sg0, sg1]
    sem_w = [sw0, sw1]
    b0 = wid * 512

    pltpu.sync_copy(x_t.at[:, pl.ds(pl.multiple_of(b0, 512), 512)],
                    idx_slab)

    def t_of(k):
        return k // BCOLS_PER_W

    def c_of(k):
        return lax.rem(k, BCOLS_PER_W)

    def start_gather(k, b):
        t = t_of(k)
        c = c_of(k)
        for l16 in range(8):
            r = idx_slab[t, pl.ds(c * 128 + l16 * 16, 16)]
            pidx[b][pl.ds(l16 * 16, 16)] = jnp.right_shift(r, 1)
        pltpu.async_copy(packed.at[pidx[b]], rows[b], sem_g[b])

    def gather_wait(b):
        pltpu.make_async_copy(packed.at[pidx[b]], rows[b], sem_g[b]).wait()

    def out_desc(k, b):
        t = t_of(k)
        c = c_of(k)
        return pltpu.make_async_copy(
            outs[b],
            out_hbm.at[pl.ds(t, 1), :,
                       pl.ds(pl.multiple_of(b0 + c * 128, 128), 128)],
            sem_w[b])

    def transpose_block(k, b):
        t = t_of(k)
        c = c_of(k)
        ov = outs[b].at[0]
        sels = []
        for l16 in range(8):
            r = idx_slab[t, pl.ds(c * 128 + l16 * 16, 16)]
            sels.append(jnp.bitwise_and(r, 1) * 64)

        def body(d, _):
            col_d = jnp.broadcast_to(d, (16,))
            for l16 in range(8):
                v = plsc.load_gather(rows[b], [_lane() + 16 * l16,
                                               sels[l16] + col_d])
                ov[d, pl.ds(l16 * 16, 16)] = v
            return 0

        lax.fori_loop(0, DIM, body, 0)

    start_gather(0, 0)

    def group(g, _):
        for b in range(2):
            k = 2 * g + b

            @pl.when(k + 1 < N_BLOCKS2)
            def _():
                start_gather(k + 1, 1 - b)

            gather_wait(b)

            @pl.when(k >= 2)
            def _():
                out_desc(k - 2, b).wait()

            transpose_block(k, b)
            out_desc(k, b).start()
        return 0

    lax.fori_loop(0, N_GROUPS2, group, 0)

    for b in range(2):
        out_desc(N_BLOCKS2 - 2 + b, b).wait()


def kernel(x, embedding):
    tail_t = jnp.pad(embedding[VOCAB - TAIL_V:].T,
                     ((0, 0), (0, 128 - TAIL_V)))
    packed = _pack_kernel(embedding.T, tail_t)
    out_t = _gather_kernel(packed, x.T)
    return out_t.transpose(2, 0, 1)


# speedup vs baseline: 1.8754x; 1.8754x over previous
import functools

import jax
import jax.numpy as jnp
from jax import lax
from jax.experimental import pallas as pl
from jax.experimental.pallas import tpu as pltpu
from jax.experimental.pallas import tpu_sc as plsc

NUM_CORES = 2
NUM_SUBCORES = 16
NUM_WORKERS = NUM_CORES * NUM_SUBCORES

BATCH = 16384
HIST_LEN = 50
DIM = 64
VOCAB = 1000000
NPAIR = VOCAB // 2
NBLK = VOCAB // 128
TAIL_V = VOCAB - NBLK * 128
TAIL_W = NBLK % NUM_WORKERS
N_GROUPS1 = 123

BCOLS = BATCH // 128
BCOLS_PER_W = BCOLS // NUM_WORKERS
N_BLOCKS2 = HIST_LEN * BCOLS_PER_W
N_GROUPS2 = N_BLOCKS2 // 2

_MESH = plsc.VectorSubcoreMesh(
    core_axis_name="c",
    subcore_axis_name="s",
    num_cores=NUM_CORES,
    num_subcores=NUM_SUBCORES,
)

_PARAMS = pltpu.CompilerParams(use_tc_tiling_on_sc=True,
                               needs_layout_passes=False)

def _lane():
    return lax.iota(jnp.int32, 16)


def _transpose_pack(in_view, out_view, width):
    @plsc.parallel_loop(0, width // 2, unroll=8)
    def _(p):
        col_a = jnp.broadcast_to(2 * p, (16,))
        col_b = col_a + 1
        for c16 in range(8):
            row_idx = _lane() + (16 * c16) % 64
            col_idx = col_a if c16 < 4 else col_b
            v = plsc.load_gather(in_view, [row_idx, col_idx])
            out_view[p, pl.ds(16 * c16, 16)] = v


@functools.partial(
    pl.kernel,
    out_type=jax.ShapeDtypeStruct((NPAIR, 128), jnp.float32),
    mesh=_MESH,
    scratch_types=(
        [pltpu.VMEM((DIM, 128), jnp.float32) for _ in range(2)]
        + [pltpu.VMEM((64, 128), jnp.float32) for _ in range(2)]
        + [pltpu.SemaphoreType.DMA for _ in range(4)]
    ),
    compiler_params=_PARAMS,
)
def _pack_kernel(tab_t, tail_t, out_hbm, in0, in1, ot0, ot1, sg0, sg1,
                 sw0, sw1):
    wid = lax.axis_index("s") * NUM_CORES + lax.axis_index("c")
    ins = [in0, in1]
    outs = [ot0, ot1]
    sem_g = [sg0, sg1]
    sem_w = [sw0, sw1]
    nk = (NBLK - 1 - wid) // NUM_WORKERS + 1

    def in_desc(k, b):
        blk = wid + k * NUM_WORKERS
        return pltpu.make_async_copy(
            tab_t.at[:, pl.ds(pl.multiple_of(blk * 128, 128), 128)],
            ins[b], sem_g[b])

    def out_desc(k, b):
        blk = wid + k * NUM_WORKERS
        return pltpu.make_async_copy(
            outs[b], out_hbm.at[pl.ds(pl.multiple_of(blk * 64, 64), 64)],
            sem_w[b])

    in_desc(0, 0).start()

    def group(g, _):
        for b in range(2):
            k = 2 * g + b

            @pl.when(k < nk)
            def _():
                @pl.when(k + 1 < nk)
                def _():
                    in_desc(k + 1, 1 - b).start()

                in_desc(k, b).wait()

                @pl.when(k >= 2)
                def _():
                    out_desc(k - 2, b).wait()

                _transpose_pack(ins[b], outs[b], 128)
                out_desc(k, b).start()
        return 0

    lax.fori_loop(0, N_GROUPS1, group, 0)

    for b in range(2):
        j = nk - 1 - lax.rem(nk - 1 + b, 2)
        out_desc(j, b).wait()

    @pl.when(wid == TAIL_W)
    def _():
        pltpu.sync_copy(tail_t, ins[0])
        _transpose_pack(ins[0], outs[0], TAIL_V)
        pltpu.sync_copy(outs[0].at[pl.ds(0, TAIL_V // 2)],
                        out_hbm.at[pl.ds(NBLK * 64, TAIL_V // 2)])


@functools.partial(
    pl.kernel,
    out_type=jax.ShapeDtypeStruct((HIST_LEN, DIM, BATCH), jnp.float32),
    mesh=_MESH,
    scratch_types=(
        [pltpu.VMEM((HIST_LEN, 512), jnp.int32)]
        + [pltpu.VMEM((128,), jnp.int32) for _ in range(2)]
        + [pltpu.VMEM((128, 128), jnp.float32) for _ in range(2)]
        + [pltpu.VMEM((1, DIM, 128), jnp.float32) for _ in range(2)]
        + [pltpu.SemaphoreType.DMA for _ in range(4)]
    ),
    compiler_params=_PARAMS,
)
def _gather_kernel(packed, x_t, out_hbm, idx_slab, pi0, pi1, rw0, rw1,
                   ot0, ot1, sg0, sg1, sw0, sw1):
    wid = lax.axis_index("s") * NUM_CORES + lax.axis_index("c")
    pidx = [pi0, pi1]
    rows = [rw0, rw1]
    outs = [ot0, ot1]
    sem_g = [sg0, sg1]
    sem_w = [sw0, sw1]
    b0 = wid * 512

    pltpu.sync_copy(x_t.at[:, pl.ds(pl.multiple_of(b0, 512), 512)],
                    idx_slab)

    def t_of(k):
        return k // BCOLS_PER_W

    def c_of(k):
        return lax.rem(k, BCOLS_PER_W)

    def start_gather(k, b):
        t = t_of(k)
        c = c_of(k)
        for l16 in range(8):
            r = idx_slab[t, pl.ds(c * 128 + l16 * 16, 16)]
            pidx[b][pl.ds(l16 * 16, 16)] = jnp.right_shift(r, 1)
        pltpu.async_copy(packed.at[pidx[b]], rows[b], sem_g[b])

    def gather_wait(b):
        pltpu.make_async_copy(packed.at[pidx[b]], rows[b], sem_g[b]).wait()

    def out_desc(k, b):
        t = t_of(k)
        c = c_of(k)
        return pltpu.make_async_copy(
            outs[b],
            out_hbm.at[pl.ds(t, 1), :,
                       pl.ds(pl.multiple_of(b0 + c * 128, 128), 128)],
            sem_w[b])

    def transpose_block(k, b):
        t = t_of(k)
        c = c_of(k)
        ov = outs[b].at[0]
        sels = []
        for l16 in range(8):
            r = idx_slab[t, pl.ds(c * 128 + l16 * 16, 16)]
            sels.append(jnp.bitwise_and(r, 1) * 64)

        @plsc.parallel_loop(0, DIM, unroll=8)
        def _(d):
            col_d = jnp.broadcast_to(d, (16,))
            for l16 in range(8):
                v = plsc.load_gather(rows[b], [_lane() + 16 * l16,
                                               sels[l16] + col_d])
                ov[d, pl.ds(l16 * 16, 16)] = v

    start_gather(0, 0)

    def group(g, _):
        for b in range(2):
            k = 2 * g + b

            @pl.when(k + 1 < N_BLOCKS2)
            def _():
                start_gather(k + 1, 1 - b)

            gather_wait(b)

            @pl.when(k >= 2)
            def _():
                out_desc(k - 2, b).wait()

            transpose_block(k, b)
            out_desc(k, b).start()
        return 0

    lax.fori_loop(0, N_GROUPS2, group, 0)

    for b in range(2):
        out_desc(N_BLOCKS2 - 2 + b, b).wait()


def kernel(x, embedding):
    tail_t = jnp.pad(embedding[VOCAB - TAIL_V:].T,
                     ((0, 0), (0, 128 - TAIL_V)))
    packed = _pack_kernel(embedding.T, tail_t)
    out_t = _gather_kernel(packed, x.T)
    return out_t.transpose(2, 0, 1)
